# baseline (device time: 29356 ns/iter reference)
import jax
import jax.numpy as jnp
from jax import lax
from jax.experimental import pallas as pl
from jax.experimental.pallas import tpu as pltpu

N_DEV = 4
SCALE = 0.08838834764831843
DH = 128
N_BLK = 2


def kernel(x, Wq, Wo, Wk, Wv):
    _, sq, d_model = x.shape
    d_local = Wq.shape[1]
    n_heads_local = d_local // DH
    x2 = x.reshape(sq, d_model)

    def body(x_ref, wq_ref, wk_ref, wv_ref, wo_ref, out_ref,
             snd1_ref, rcv1_ref, snd2_ref, rcv2_ref, send_sems, recv_sems):
        my = lax.axis_index("i")
        left = lax.rem(my + N_DEV - 1, N_DEV)
        right = lax.rem(my + 1, N_DEV)

        barrier = pltpu.get_barrier_semaphore()
        for nbr in (left, right):
            pl.semaphore_signal(
                barrier, inc=1,
                device_id=(nbr,), device_id_type=pl.DeviceIdType.MESH,
            )
        pl.semaphore_wait(barrier, 2)

        xb = x_ref[...]
        wqb = wq_ref[...]
        wob = wo_ref[...]
        k = jnp.dot(xb, wk_ref[...], preferred_element_type=jnp.float32)
        v = jnp.dot(xb, wv_ref[...], preferred_element_type=jnp.float32)

        n_blk = N_BLK
        rows = sq // n_blk
        pa = jnp.bitwise_xor(my, 1)
        pb = 3 - my

        x1s = []
        partials = []
        for b in range(n_blk):
            r0 = b * rows
            qb = jnp.dot(xb[r0:r0 + rows, :], wqb,
                         preferred_element_type=jnp.float32)
            head_outs = []
            for h in range(n_heads_local):
                qh = qb[:, h * DH:(h + 1) * DH]
                kh = k[:, h * DH:(h + 1) * DH]
                vh = v[:, h * DH:(h + 1) * DH]
                s = lax.dot_general(
                    qh, kh, (((1,), (1,)), ((), ())),
                    preferred_element_type=jnp.float32,
                ) * SCALE
                m = jnp.max(s, axis=1, keepdims=True)
                p = jnp.exp(s - m)
                l = jnp.sum(p, axis=1, keepdims=True)
                o = jnp.dot(p, vh, preferred_element_type=jnp.float32)
                head_outs.append(o / l)
            attn_b = jnp.concatenate(head_outs, axis=1)
            p_b = jnp.dot(attn_b, wob, preferred_element_type=jnp.float32)
            partials.append(p_b)
            snd1_ref[r0:r0 + rows, :] = p_b.astype(jnp.bfloat16)

            x1 = pltpu.make_async_remote_copy(
                src_ref=snd1_ref.at[r0:r0 + rows, :],
                dst_ref=rcv1_ref.at[r0:r0 + rows, :],
                send_sem=send_sems.at[2 * b + 0],
                recv_sem=recv_sems.at[2 * b + 0],
                device_id=(pa,), device_id_type=pl.DeviceIdType.MESH,
            )
            x1.start()
            x1s.append(x1)

        x2s = []
        sums2 = []
        for b in range(n_blk):
            r0 = b * rows
            x1s[b].wait_recv()
            s2 = partials[b] + rcv1_ref[r0:r0 + rows, :].astype(jnp.float32)
            sums2.append(s2)
            snd2_ref[r0:r0 + rows, :] = s2.astype(jnp.bfloat16)
            x2 = pltpu.make_async_remote_copy(
                src_ref=snd2_ref.at[r0:r0 + rows, :],
                dst_ref=rcv2_ref.at[r0:r0 + rows, :],
                send_sem=send_sems.at[2 * b + 1],
                recv_sem=recv_sems.at[2 * b + 1],
                device_id=(pb,), device_id_type=pl.DeviceIdType.MESH,
            )
            x2.start()
            x2s.append(x2)

        for b in range(n_blk):
            r0 = b * rows
            x2s[b].wait_recv()
            out_ref[r0:r0 + rows, :] = (
                sums2[b] + rcv2_ref[r0:r0 + rows, :].astype(jnp.float32)
            )

        for b in range(n_blk):
            x1s[b].wait_send()
            x2s[b].wait_send()

    out = pl.pallas_call(
        body,
        out_shape=jax.ShapeDtypeStruct((sq, d_model), jnp.float32),
        in_specs=[pl.BlockSpec(memory_space=pltpu.VMEM)] * 5,
        out_specs=pl.BlockSpec(memory_space=pltpu.VMEM),
        scratch_shapes=[
            pltpu.VMEM((sq, d_model), jnp.bfloat16),
            pltpu.VMEM((sq, d_model), jnp.bfloat16),
            pltpu.VMEM((sq, d_model), jnp.bfloat16),
            pltpu.VMEM((sq, d_model), jnp.bfloat16),
            pltpu.SemaphoreType.DMA((2 * N_BLK,)),
            pltpu.SemaphoreType.DMA((2 * N_BLK,)),
        ],
        compiler_params=pltpu.CompilerParams(collective_id=0),
    )(x2, Wq, Wk, Wv, Wo)
    return out.reshape(1, sq, d_model)
